# baseline (speedup 1.0000x reference)
import functools

import jax
import jax.numpy as jnp
from jax import lax
from jax.experimental import pallas as pl
from jax.experimental.pallas import tpu as pltpu
from jax.experimental.pallas import tpu_sc as plsc

B = 32
T = 2048
N = B * T
D0, D1 = 128, 64
W, C = 64, 5
R2 = C * W
CLOSE_IDX = 3
NC, NS = 2, 16

LOGT = 11
NVREG = T // 16
G = 8


def _make_kernel():
    mesh = plsc.VectorSubcoreMesh(core_axis_name="c", subcore_axis_name="s",
                                  num_cores=NC, num_subcores=NS)
    out_type = (
        jax.ShapeDtypeStruct((N, D0), jnp.float32),
        jax.ShapeDtypeStruct((B * D1, T), jnp.float32),
        jax.ShapeDtypeStruct((B * R2, T), jnp.float32),
        jax.ShapeDtypeStruct((B, T), jnp.int32),
    )
    scratch = [
        pltpu.VMEM((T,), jnp.int32),
        pltpu.VMEM((T,), jnp.int32),
        pltpu.VMEM((T,), jnp.int32),
        pltpu.VMEM((T,), jnp.int32),
        pltpu.VMEM((4096,), jnp.int32),
        pltpu.VMEM((T,), jnp.int32),
        pltpu.VMEM((2, 128), jnp.int32),
        pltpu.VMEM((2, 128, D0), jnp.float32),
        pltpu.VMEM((G, T), jnp.float32),
        pltpu.VMEM((G, T), jnp.float32),
        pltpu.VMEM((G, T), jnp.float32),
        pltpu.VMEM((G, T), jnp.float32),
        pltpu.SemaphoreType.DMA,
        pltpu.SemaphoreType.DMA,
        pltpu.SemaphoreType.DMA,
        pltpu.SemaphoreType.DMA,
    ]

    def body(in0f, in1t, in2t, out0, out1t, out2t, orders,
             PKa, PKb, PVa, PVb, hist, V, idx0, rbuf, bin0, bin1, bout0, bout1,
             gsem0, gsem1, wsem0, wsem1):
        cid = lax.axis_index("c")
        sid = lax.axis_index("s")
        b = sid * NC + cid
        lane = lax.iota(jnp.int32, 16)
        base_t = b * T

        row_s = b * R2 + CLOSE_IDX * W
        row_l = b * R2 + CLOSE_IDX * W + W - G
        h1 = pltpu.async_copy(in2t.at[pl.ds(row_s, G)], bin0, gsem0)
        h2 = pltpu.async_copy(in2t.at[pl.ds(row_l, G)], bin1, gsem1)
        h1.wait()
        h2.wait()

        zvec = jnp.broadcast_to(jnp.int32(0), (16,))
        gvec = jnp.broadcast_to(jnp.int32(G - 1), (16,))

        def key_body(i, _):
            tvec = lane * 128 + i
            s = plsc.load_gather(bin0, [zvec, tvec])
            last = plsc.load_gather(bin1, [gvec, tvec])
            nz = s != 0.0
            safe = jnp.where(nz, s, jnp.float32(1.0))
            perf = jnp.where(nz, (last - s) / safe, jnp.float32(0.0))
            f = (-perf) + jnp.float32(0.0)
            u = plsc.bitcast(f, jnp.int32)
            ku = jnp.where(u < 0, ~u, u | jnp.int32(-2**31))
            PKa[pl.ds(i * 16, 16)] = ku
            PVa[pl.ds(i * 16, 16)] = tvec
            return 0

        with jax.named_scope("p2_keys"):
            lax.fori_loop(0, NVREG, key_body, 0)

        ones = jnp.broadcast_to(jnp.int32(1), (16,))

        for pas in range(4):
            sh = 8 * pas
            src_k, src_v = (PKa, PVa) if pas % 2 == 0 else (PKb, PVb)
            dst_k, dst_v = (PKb, PVb) if pas % 2 == 0 else (PKa, PVa)

            def zero_body(h, _):
                hist[pl.ds(h * 16, 16)] = zvec
                return 0

            lax.fori_loop(0, 256, zero_body, 0)

            def count_body(i, _, src_k=src_k, sh=sh):
                kv = src_k[pl.ds(i * 16, 16)]
                d = lax.shift_right_logical(kv, sh) & 255
                plsc.addupdate_scatter(hist, [d * 16 + lane], ones)
                return 0

            with jax.named_scope("p3_radix_count"):
                lax.fori_loop(0, NVREG, count_body, 0)

            def scan_body(d, run):
                row = hist[pl.ds(d * 16, 16)]
                csum = plsc.cumsum(row)
                hist[pl.ds(d * 16, 16)] = (csum - row) + run
                return run + csum[15]

            lax.fori_loop(0, 256, scan_body, jnp.int32(0))

            def perm_body(i, _, src_k=src_k, src_v=src_v,
                          dst_k=dst_k, dst_v=dst_v, sh=sh, last=(pas == 3)):
                kv = src_k[pl.ds(i * 16, 16)]
                vv = src_v[pl.ds(i * 16, 16)]
                d = lax.shift_right_logical(kv, sh) & 255
                hidx = d * 16 + lane
                pos = plsc.load_gather(hist, [hidx])
                plsc.store_scatter(hist, [hidx], pos + 1)
                if last:
                    plsc.store_scatter(V, [pos], vv)
                else:
                    pi = ((pos & 127) << 4) | lax.shift_right_logical(pos, 7)
                    plsc.store_scatter(dst_k, [pi], kv)
                    plsc.store_scatter(dst_v, [pi], vv)
                return 0

            with jax.named_scope("p3_radix_pass"):
                lax.fori_loop(0, NVREG, perm_body, 0)

        with jax.named_scope("p4_orders"):
            pltpu.sync_copy(V, orders.at[b])

        gsems = (gsem0, gsem1)
        wsems = (wsem0, wsem1)
        nch = T // 128

        def in0_phase():
            gh = [None, None]
            wh = [None, None]
            for c in range(nch + 1):
                if c < nch:
                    slot = c % 2
                    if wh[slot] is not None:
                        wh[slot].wait()
                    for sub in range(8):
                        t = V[pl.ds(c * 128 + sub * 16, 16)]
                        idx0[slot, pl.ds(sub * 16, 16)] = t + base_t
                    gh[slot] = pltpu.async_copy(
                        in0f.at[idx0.at[slot]], rbuf.at[slot], gsems[slot])
                if c >= 1:
                    pslot = (c - 1) % 2
                    gh[pslot].wait()
                    wh[pslot] = pltpu.async_copy(
                        rbuf.at[pslot],
                        out0.at[pl.ds(base_t + (c - 1) * 128, 128)],
                        wsems[pslot])
            for slot in (0, 1):
                if wh[slot] is not None:
                    wh[slot].wait()

        with jax.named_scope("p5a_in0"):
            in0_phase()

        bins = (bin0, bin1)
        bouts = (bout0, bout1)

        def permute_minor(int_ref, outt_ref, rowbase, ngroups):
            gh = [None, None]
            wh = [None, None]

            def grp_body(bi, bo):
                def chunk_body(i, _):
                    v16 = V[pl.ds(i * 16, 16)]
                    for r in range(G):
                        rvec = jnp.broadcast_to(jnp.int32(r), (16,))
                        x = plsc.load_gather(bi, [rvec, v16])
                        bo[r, pl.ds(i * 16, 16)] = x
                    return 0
                lax.fori_loop(0, NVREG, chunk_body, 0)

            for g in range(ngroups + 1):
                if g < ngroups:
                    slot = g % 2
                    gh[slot] = pltpu.async_copy(
                        int_ref.at[pl.ds(rowbase + g * G, G)],
                        bins[slot], gsems[slot])
                if g >= 1:
                    pslot = (g - 1) % 2
                    gh[pslot].wait()
                    if wh[pslot] is not None:
                        wh[pslot].wait()
                    grp_body(bins[pslot], bouts[pslot])
                    wh[pslot] = pltpu.async_copy(
                        bouts[pslot],
                        outt_ref.at[pl.ds(rowbase + (g - 1) * G, G)],
                        wsems[pslot])
            for slot in (0, 1):
                if wh[slot] is not None:
                    wh[slot].wait()

        with jax.named_scope("p5b_in1"):
            permute_minor(in1t, out1t, b * D1, D1 // G)
        with jax.named_scope("p5c_in2"):
            permute_minor(in2t, out2t, b * R2, R2 // G)

    return pl.kernel(
        body, out_type=out_type, mesh=mesh, scratch_types=scratch,
        compiler_params=pltpu.CompilerParams(needs_layout_passes=False))


@functools.cache
def _arranger():
    return _make_kernel()


def kernel(in0, in1, in2):
    in0f = in0.reshape(N, D0)
    in1t = jnp.transpose(in1, (0, 2, 1)).reshape(B * D1, T)
    in2t = jnp.transpose(in2, (0, 3, 2, 1)).reshape(B * R2, T)
    out0, out1t, out2t, orders = _arranger()(in0f, in1t, in2t)
    out1 = jnp.transpose(out1t.reshape(B, D1, T), (0, 2, 1))
    out2 = jnp.transpose(out2t.reshape(B, C, W, T), (0, 3, 2, 1))
    return (out0.reshape(B, T, D0), out1, out2, orders)


# speedup vs baseline: 1.5199x; 1.5199x over previous
import functools

import jax
import jax.numpy as jnp
from jax import lax
from jax.experimental import pallas as pl
from jax.experimental.pallas import tpu as pltpu
from jax.experimental.pallas import tpu_sc as plsc

B = 32
T = 2048
N = B * T
D0, D1 = 128, 64
W, C = 64, 5
R2 = C * W
CLOSE_IDX = 3
NC, NS = 2, 16

LOGT = 11
NVREG = T // 16
G = 8


def _make_kernel():
    mesh = plsc.VectorSubcoreMesh(core_axis_name="c", subcore_axis_name="s",
                                  num_cores=NC, num_subcores=NS)
    out_type = (
        jax.ShapeDtypeStruct((N, D0), jnp.float32),
        jax.ShapeDtypeStruct((B * D1, T), jnp.float32),
        jax.ShapeDtypeStruct((B * R2, T), jnp.float32),
        jax.ShapeDtypeStruct((B, T), jnp.int32),
    )
    scratch = [
        pltpu.VMEM((T,), jnp.int32),
        pltpu.VMEM((T,), jnp.int32),
        pltpu.VMEM((T,), jnp.int32),
        pltpu.VMEM((T,), jnp.int32),
        pltpu.VMEM((4096,), jnp.int32),
        pltpu.VMEM((T,), jnp.int32),
        pltpu.VMEM((2, 128), jnp.int32),
        pltpu.VMEM((2, 128, D0), jnp.float32),
        pltpu.VMEM((G, T), jnp.float32),
        pltpu.VMEM((G, T), jnp.float32),
        pltpu.VMEM((G, T), jnp.float32),
        pltpu.VMEM((G, T), jnp.float32),
        pltpu.SemaphoreType.DMA,
        pltpu.SemaphoreType.DMA,
        pltpu.SemaphoreType.DMA,
        pltpu.SemaphoreType.DMA,
    ]

    def body(in0f, in1t, in2t, out0, out1t, out2t, orders,
             PKa, PKb, PVa, PVb, hist, V, idx0, rbuf, bin0, bin1, bout0, bout1,
             gsem0, gsem1, wsem0, wsem1):
        cid = lax.axis_index("c")
        sid = lax.axis_index("s")
        b = sid * NC + cid
        lane = lax.iota(jnp.int32, 16)
        base_t = b * T

        row_s = b * R2 + CLOSE_IDX * W
        row_l = b * R2 + CLOSE_IDX * W + W - G
        h1 = pltpu.async_copy(in2t.at[pl.ds(row_s, G)], bin0, gsem0)
        h2 = pltpu.async_copy(in2t.at[pl.ds(row_l, G)], bin1, gsem1)
        h1.wait()
        h2.wait()

        zvec = jnp.broadcast_to(jnp.int32(0), (16,))
        gvec = jnp.broadcast_to(jnp.int32(G - 1), (16,))

        def key_body(i, _):
            tvec = lane * 128 + i
            s = plsc.load_gather(bin0, [zvec, tvec])
            last = plsc.load_gather(bin1, [gvec, tvec])
            nz = s != 0.0
            safe = jnp.where(nz, s, jnp.float32(1.0))
            perf = jnp.where(nz, (last - s) / safe, jnp.float32(0.0))
            f = (-perf) + jnp.float32(0.0)
            u = plsc.bitcast(f, jnp.int32)
            ku = jnp.where(u < 0, ~u, u | jnp.int32(-2**31))
            PKa[pl.ds(i * 16, 16)] = ku
            PVa[pl.ds(i * 16, 16)] = tvec
            return 0

        with jax.named_scope("p2_keys"):
            lax.fori_loop(0, NVREG, key_body, 0)

        ones = jnp.broadcast_to(jnp.int32(1), (16,))

        for pas in range(4):
            sh = 8 * pas
            src_k, src_v = (PKa, PVa) if pas % 2 == 0 else (PKb, PVb)
            dst_k, dst_v = (PKb, PVb) if pas % 2 == 0 else (PKa, PVa)

            def zero_body(h, _):
                hist[pl.ds(h * 16, 16)] = zvec
                return 0

            lax.fori_loop(0, 256, zero_body, 0)

            def count_body(i, _, src_k=src_k, sh=sh):
                kv = src_k[pl.ds(i * 16, 16)]
                d = lax.shift_right_logical(kv, sh) & 255
                plsc.addupdate_scatter(hist, [d * 16 + lane], ones)
                return 0

            with jax.named_scope("p3_radix_count"):
                lax.fori_loop(0, NVREG, count_body, 0)

            def scan_body(d, run):
                row = hist[pl.ds(d * 16, 16)]
                csum = plsc.cumsum(row)
                hist[pl.ds(d * 16, 16)] = (csum - row) + run
                return run + csum[15]

            lax.fori_loop(0, 256, scan_body, jnp.int32(0))

            def perm_body(i, _, src_k=src_k, src_v=src_v,
                          dst_k=dst_k, dst_v=dst_v, sh=sh, last=(pas == 3)):
                kv = src_k[pl.ds(i * 16, 16)]
                vv = src_v[pl.ds(i * 16, 16)]
                d = lax.shift_right_logical(kv, sh) & 255
                hidx = d * 16 + lane
                pos = plsc.load_gather(hist, [hidx])
                plsc.store_scatter(hist, [hidx], pos + 1)
                if last:
                    plsc.store_scatter(V, [pos], vv)
                else:
                    pi = ((pos & 127) << 4) | lax.shift_right_logical(pos, 7)
                    plsc.store_scatter(dst_k, [pi], kv)
                    plsc.store_scatter(dst_v, [pi], vv)
                return 0

            with jax.named_scope("p3_radix_pass"):
                lax.fori_loop(0, NVREG, perm_body, 0)

        with jax.named_scope("p4_orders"):
            pltpu.sync_copy(V, orders.at[b])

        gsems = (gsem0, gsem1)
        wsems = (wsem0, wsem1)
        nch = T // 128

        def in0_phase():
            gh = [None, None]
            wh = [None, None]
            for c in range(nch + 1):
                if c < nch:
                    slot = c % 2
                    if wh[slot] is not None:
                        wh[slot].wait()
                    for sub in range(8):
                        t = V[pl.ds(c * 128 + sub * 16, 16)]
                        idx0[slot, pl.ds(sub * 16, 16)] = t + base_t
                    gh[slot] = pltpu.async_copy(
                        in0f.at[idx0.at[slot]], rbuf.at[slot], gsems[slot])
                if c >= 1:
                    pslot = (c - 1) % 2
                    gh[pslot].wait()
                    wh[pslot] = pltpu.async_copy(
                        rbuf.at[pslot],
                        out0.at[pl.ds(base_t + (c - 1) * 128, 128)],
                        wsems[pslot])
            for slot in (0, 1):
                if wh[slot] is not None:
                    wh[slot].wait()

        with jax.named_scope("p5a_in0"):
            in0_phase()

        bins = (bin0, bin1)
        bouts = (bout0, bout1)

        def permute_minor(int_ref, outt_ref, rowbase, ngroups):
            gh = [None, None]
            wh = [None, None]

            def grp_body(bi, bo):
                def chunk_body(i, _):
                    v16 = V[pl.ds(i * 16, 16)]
                    xs = [plsc.load_gather(
                              bi, [jnp.broadcast_to(jnp.int32(r), (16,)), v16])
                          for r in range(G)]
                    for r in range(G):
                        bo[r, pl.ds(i * 16, 16)] = xs[r]
                    return 0
                lax.fori_loop(0, NVREG, chunk_body, 0)

            for g in range(ngroups + 1):
                if g < ngroups:
                    slot = g % 2
                    gh[slot] = pltpu.async_copy(
                        int_ref.at[pl.ds(rowbase + g * G, G)],
                        bins[slot], gsems[slot])
                if g >= 1:
                    pslot = (g - 1) % 2
                    gh[pslot].wait()
                    if wh[pslot] is not None:
                        wh[pslot].wait()
                    grp_body(bins[pslot], bouts[pslot])
                    wh[pslot] = pltpu.async_copy(
                        bouts[pslot],
                        outt_ref.at[pl.ds(rowbase + (g - 1) * G, G)],
                        wsems[pslot])
            for slot in (0, 1):
                if wh[slot] is not None:
                    wh[slot].wait()

        with jax.named_scope("p5b_in1"):
            permute_minor(in1t, out1t, b * D1, D1 // G)
        with jax.named_scope("p5c_in2"):
            permute_minor(in2t, out2t, b * R2, R2 // G)

    return pl.kernel(
        body, out_type=out_type, mesh=mesh, scratch_types=scratch,
        compiler_params=pltpu.CompilerParams(needs_layout_passes=False))


@functools.cache
def _arranger():
    return _make_kernel()


def kernel(in0, in1, in2):
    in0f = in0.reshape(N, D0)
    in1t = jnp.transpose(in1, (0, 2, 1)).reshape(B * D1, T)
    in2t = jnp.transpose(in2, (0, 3, 2, 1)).reshape(B * R2, T)
    out0, out1t, out2t, orders = _arranger()(in0f, in1t, in2t)
    out1 = jnp.transpose(out1t.reshape(B, D1, T), (0, 2, 1))
    out2 = jnp.transpose(out2t.reshape(B, C, W, T), (0, 3, 2, 1))
    return (out0.reshape(B, T, D0), out1, out2, orders)
